# baseline (speedup 1.0000x reference)
import functools

import jax
import jax.numpy as jnp
from jax import lax
from jax.experimental import pallas as pl
from jax.experimental.pallas import tpu as pltpu
from jax.experimental.pallas import tpu_sc as plsc

N = 10000
F = 256
H = 512
C = 64
R = 10240
S = 4
SW = 128
E = 160000
NT = 16
ECH = 128
NCH = 80
EPAD = NT * NCH * ECH
HR = R // 2
HRPT = HR // NT
RPT = R // NT
BR = 2048
NRB = R // BR

_PREC = jax.lax.Precision.HIGHEST


def _mesh():
    return plsc.VectorSubcoreMesh(core_axis_name="c", subcore_axis_name="s")


def _sc_degree(dsth, zeros128, ones128):
    @functools.partial(
        pl.kernel,
        out_type=jax.ShapeDtypeStruct((R, 128), jnp.float32),
        mesh=_mesh(),
        scratch_types=[
            pltpu.VMEM((NCH, ECH), jnp.int32),
            pltpu.VMEM((ECH, 128), jnp.float32),
            pltpu.VMEM_SHARED((HR + 8, 128), jnp.float32),
        ],
    )
    def k(dst_hbm, z_hbm, one_hbm, out_hbm, didx, ones_v, acc):
        c = lax.axis_index("c")
        t = lax.axis_index("s")
        pltpu.sync_copy(one_hbm, ones_v)
        pltpu.sync_copy(z_hbm.at[pl.ds(t * HRPT, HRPT)],
                        acc.at[pl.ds(t * HRPT, HRPT)])
        pltpu.sync_copy(dst_hbm.at[c, t], didx)
        plsc.subcore_barrier()

        @pl.loop(0, NCH)
        def _(i):
            pltpu.sync_copy(ones_v, acc.at[didx.at[i]], add=True)

        plsc.subcore_barrier()
        pltpu.sync_copy(acc.at[pl.ds(t * HRPT, HRPT)],
                        out_hbm.at[pl.ds(c * HR + t * HRPT, HRPT)])

    return k(dsth, zeros128, ones128)


def _sc_aggregate(g, src_slab, dstp):
    @functools.partial(
        pl.kernel,
        out_type=jax.ShapeDtypeStruct((S * R, SW), jnp.float32),
        mesh=_mesh(),
        scratch_types=[
            pltpu.VMEM((NCH, ECH), jnp.int32),
            pltpu.VMEM((NCH, ECH), jnp.int32),
            pltpu.VMEM((ECH, SW), jnp.float32),
            pltpu.VMEM_SHARED((R, SW), jnp.float32),
            pltpu.SemaphoreType.DMA,
        ],
    )
    def k(g_hbm, src_hbm, dst_hbm, out_hbm, sidx, didx, gbuf, acc, sem):
        c = lax.axis_index("c")
        t = lax.axis_index("s")
        pltpu.sync_copy(dst_hbm.at[t], didx)
        for j in range(S // 2):
            s = c * (S // 2) + j
            pltpu.sync_copy(src_hbm.at[s, t], sidx)
            pltpu.sync_copy(g_hbm.at[pl.ds(s * R + t * RPT, RPT)],
                            acc.at[pl.ds(t * RPT, RPT)])
            plsc.subcore_barrier()

            @pl.loop(0, NCH)
            def _(i):
                pltpu.async_copy(g_hbm.at[sidx.at[i]], gbuf, sem).wait()
                pltpu.sync_copy(gbuf, acc.at[didx.at[i]], add=True)

            plsc.subcore_barrier()
            pltpu.sync_copy(acc.at[pl.ds(t * RPT, RPT)],
                            out_hbm.at[pl.ds(s * R + t * RPT, RPT)])
            plsc.subcore_barrier()

    return k(g, src_slab, dstp)


def _dot(a, b):
    return jax.lax.dot_general(a, b, (((1,), (0,)), ((), ())),
                               precision=_PREC,
                               preferred_element_type=jnp.float32)


def _tc_dinv(degp):
    def body(a_ref, o_ref):
        i = pl.program_id(0)
        deg = 1.0 + a_ref[:, :1]
        rows = jax.lax.broadcasted_iota(jnp.int32, (BR, 1), 0) + i * BR
        dv = jnp.where(rows < N, jax.lax.rsqrt(deg), 0.0)
        o_ref[...] = jnp.broadcast_to(dv, (BR, 128))

    return pl.pallas_call(
        body,
        grid=(NRB,),
        in_specs=[pl.BlockSpec((BR, 128), lambda i: (i, 0))],
        out_specs=pl.BlockSpec((BR, 128), lambda i: (i, 0)),
        out_shape=jax.ShapeDtypeStruct((R, 128), jnp.float32),
    )(degp)


def _tc_wprod(a, b):
    def body(a_ref, b_ref, o_ref):
        o_ref[...] = _dot(a_ref[...], b_ref[...])

    return pl.pallas_call(
        body, out_shape=jax.ShapeDtypeStruct((F, H), jnp.float32),
    )(a, b)


def _tc_first(xp, Wc, dinv):
    def body(x_ref, w_ref, d_ref, o_ref):
        o_ref[...] = d_ref[:, :1] * _dot(x_ref[...], w_ref[...])

    return pl.pallas_call(
        body,
        grid=(S, NRB),
        in_specs=[pl.BlockSpec((BR, F), lambda s, i: (i, 0)),
                  pl.BlockSpec((F, SW), lambda s, i: (0, s)),
                  pl.BlockSpec((BR, 128), lambda s, i: (i, 0))],
        out_specs=pl.BlockSpec((BR, SW), lambda s, i: (s * NRB + i, 0)),
        out_shape=jax.ShapeDtypeStruct((S * R, SW), jnp.float32),
    )(xp, Wc, dinv)


def _tc_mid(agg, b3, W, dinv):
    def body(a_ref, b_ref, w_ref, d_ref, o_ref):
        k = pl.program_id(2)
        dv = d_ref[:, :1]
        hblk = jnp.maximum(dv * a_ref[...] + b_ref[0], 0.0)
        mm = _dot(hblk, w_ref[...])

        @pl.when(k == 0)
        def _():
            o_ref[...] = mm

        @pl.when(k > 0)
        def _():
            o_ref[...] += mm

        @pl.when(k == S - 1)
        def _():
            o_ref[...] *= dv

    return pl.pallas_call(
        body,
        grid=(S, NRB, S),
        in_specs=[pl.BlockSpec((BR, SW), lambda s, i, k: (k * NRB + i, 0)),
                  pl.BlockSpec((1, 1, SW), lambda s, i, k: (k, 0, 0)),
                  pl.BlockSpec((SW, SW), lambda s, i, k: (k, s)),
                  pl.BlockSpec((BR, 128), lambda s, i, k: (i, 0))],
        out_specs=pl.BlockSpec((BR, SW), lambda s, i, k: (s * NRB + i, 0)),
        out_shape=jax.ShapeDtypeStruct((S * R, SW), jnp.float32),
    )(agg, b3, W, dinv)


def _tc_cls(agg, b3, Wout, dinv):
    def body(a_ref, b_ref, w_ref, d_ref, o_ref):
        k = pl.program_id(1)
        dv = d_ref[:, :1]
        hblk = jnp.maximum(dv * a_ref[...] + b_ref[0], 0.0)
        mm = _dot(hblk, w_ref[...])

        @pl.when(k == 0)
        def _():
            o_ref[...] = mm

        @pl.when(k > 0)
        def _():
            o_ref[...] += mm

        @pl.when(k == S - 1)
        def _():
            z = o_ref[...]
            m = jnp.max(z, axis=1, keepdims=True)
            ez = jnp.exp(z - m)
            o_ref[...] = (z - m) - jnp.log(jnp.sum(ez, axis=1, keepdims=True))

    return pl.pallas_call(
        body,
        grid=(NRB, S),
        in_specs=[pl.BlockSpec((BR, SW), lambda i, k: (k * NRB + i, 0)),
                  pl.BlockSpec((1, 1, SW), lambda i, k: (k, 0, 0)),
                  pl.BlockSpec((SW, C), lambda i, k: (k, 0)),
                  pl.BlockSpec((BR, 128), lambda i, k: (i, 0))],
        out_specs=pl.BlockSpec((BR, C), lambda i, k: (i, 0)),
        out_shape=jax.ShapeDtypeStruct((R, C), jnp.float32),
    )(agg, b3, Wout, dinv)


def kernel(x, edge_index, weight_in, weight_out, W0, W1, W2, b0, b1, b2):
    i32 = jnp.int32
    f32 = jnp.float32
    src = edge_index[0]
    dst = edge_index[1]

    pad = jnp.full((EPAD - E,), N, i32)
    srcp = jnp.concatenate([src, pad]).reshape(NT, NCH, ECH)
    dstp = jnp.concatenate([dst, pad]).reshape(NT, NCH, ECH)
    src_slab = srcp[None] + (jnp.arange(S, dtype=i32) * R)[:, None, None, None]
    dflat = jnp.concatenate([dst, pad])
    dsth = jnp.stack([
        jnp.where((dflat >= c * HR) & (dflat < (c + 1) * HR),
                  dflat - c * HR, HR)
        for c in range(2)
    ]).reshape(2, NT, NCH, ECH)
    xp = jnp.pad(x, ((0, R - N), (0, 0)))
    zeros128 = jnp.zeros((R, 128), f32)
    ones128 = jnp.ones((ECH, 128), f32)

    degp = _sc_degree(dsth, zeros128, ones128)
    dinv = _tc_dinv(degp)
    Wc = _tc_wprod(weight_in, W0)
    g = _tc_first(xp, Wc, dinv)
    for (W, b) in ((W1, b0), (W2, b1)):
        agg = _sc_aggregate(g, src_slab, dstp)
        g = _tc_mid(agg, b.reshape(S, 1, SW), W, dinv)
    agg = _sc_aggregate(g, src_slab, dstp)
    out = _tc_cls(agg, b2.reshape(S, 1, SW), weight_out, dinv)
    return out[:N]


# speedup vs baseline: 1.4386x; 1.4386x over previous
import functools

import jax
import jax.numpy as jnp
from jax import lax
from jax.experimental import pallas as pl
from jax.experimental.pallas import tpu as pltpu
from jax.experimental.pallas import tpu_sc as plsc

N = 10000
F = 256
H = 512
C = 64
R = 10240
S = 4
SW = 128
E = 160000
NT = 16
ECH = 128
NCH = 79
EPAD = NT * NCH * ECH
NW = 32
NCH2 = 40
EPAD2 = NW * NCH2 * ECH
RPT = R // NT
BR = 2048
NRB = R // BR

_PREC = jax.lax.Precision.HIGHEST


def _mesh():
    return plsc.VectorSubcoreMesh(core_axis_name="c", subcore_axis_name="s")


def _sc_degree(dst2, zeros128, ones128):
    @functools.partial(
        pl.kernel,
        out_type=jax.ShapeDtypeStruct((2 * R, 128), jnp.float32),
        mesh=_mesh(),
        scratch_types=[
            pltpu.VMEM((NCH2, ECH), jnp.int32),
            pltpu.VMEM((ECH, 128), jnp.float32),
            pltpu.VMEM_SHARED((R, 128), jnp.float32),
        ],
    )
    def k(dst_hbm, z_hbm, one_hbm, out_hbm, didx, ones_v, acc):
        c = lax.axis_index("c")
        t = lax.axis_index("s")
        pltpu.sync_copy(one_hbm, ones_v)
        pltpu.sync_copy(z_hbm.at[pl.ds(t * RPT, RPT)], acc.at[pl.ds(t * RPT, RPT)])
        pltpu.sync_copy(dst_hbm.at[c, t], didx)
        plsc.subcore_barrier()

        @pl.loop(0, NCH2)
        def _(i):
            pltpu.sync_copy(ones_v, acc.at[didx.at[i]], add=True)

        plsc.subcore_barrier()
        pltpu.sync_copy(acc.at[pl.ds(t * RPT, RPT)],
                        out_hbm.at[pl.ds(c * R + t * RPT, RPT)])

    return k(dst2, zeros128, ones128)


def _sc_aggregate(g, src_slab, dstp):
    @functools.partial(
        pl.kernel,
        out_type=jax.ShapeDtypeStruct((S * R, SW), jnp.float32),
        mesh=_mesh(),
        scratch_types=[
            pltpu.VMEM((NCH, ECH), jnp.int32),
            pltpu.VMEM((NCH, ECH), jnp.int32),
            pltpu.VMEM((ECH, SW), jnp.float32),
            pltpu.VMEM_SHARED((R, SW), jnp.float32),
            pltpu.SemaphoreType.DMA,
        ],
    )
    def k(g_hbm, src_hbm, dst_hbm, out_hbm, sidx, didx, gbuf, acc, sem):
        c = lax.axis_index("c")
        t = lax.axis_index("s")
        pltpu.sync_copy(dst_hbm.at[t], didx)
        for j in range(S // 2):
            s = c * (S // 2) + j
            pltpu.sync_copy(src_hbm.at[s, t], sidx)
            pltpu.sync_copy(g_hbm.at[pl.ds(s * R + t * RPT, RPT)],
                            acc.at[pl.ds(t * RPT, RPT)])
            plsc.subcore_barrier()

            @pl.loop(0, NCH)
            def _(i):
                pltpu.async_copy(g_hbm.at[sidx.at[i]], gbuf, sem).wait()
                pltpu.sync_copy(gbuf, acc.at[didx.at[i]], add=True)

            plsc.subcore_barrier()
            pltpu.sync_copy(acc.at[pl.ds(t * RPT, RPT)],
                            out_hbm.at[pl.ds(s * R + t * RPT, RPT)])

    return k(g, src_slab, dstp)


def _dot(a, b):
    return jax.lax.dot_general(a, b, (((1,), (0,)), ((), ())),
                               precision=_PREC,
                               preferred_element_type=jnp.float32)


def _tc_dinv(degp):
    def body(a_ref, b_ref, o_ref):
        i = pl.program_id(0)
        deg = 1.0 + a_ref[:, :1] + b_ref[:, :1]
        rows = jax.lax.broadcasted_iota(jnp.int32, (BR, 1), 0) + i * BR
        dv = jnp.where(rows < N, jax.lax.rsqrt(deg), 0.0)
        o_ref[...] = jnp.broadcast_to(dv, (BR, 128))

    return pl.pallas_call(
        body,
        grid=(NRB,),
        in_specs=[pl.BlockSpec((BR, 128), lambda i: (i, 0)),
                  pl.BlockSpec((BR, 128), lambda i: (i + NRB, 0))],
        out_specs=pl.BlockSpec((BR, 128), lambda i: (i, 0)),
        out_shape=jax.ShapeDtypeStruct((R, 128), jnp.float32),
    )(degp, degp)


def _tc_wprod(a, b):
    def body(a_ref, b_ref, o_ref):
        o_ref[...] = _dot(a_ref[...], b_ref[...])

    return pl.pallas_call(
        body, out_shape=jax.ShapeDtypeStruct((F, H), jnp.float32),
    )(a, b)


def _tc_first(xp, Wc, dinv):
    def body(x_ref, w_ref, d_ref, o_ref):
        o_ref[...] = d_ref[:, :1] * _dot(x_ref[...], w_ref[...])

    return pl.pallas_call(
        body,
        grid=(S, NRB),
        in_specs=[pl.BlockSpec((BR, F), lambda s, i: (i, 0)),
                  pl.BlockSpec((F, SW), lambda s, i: (0, s)),
                  pl.BlockSpec((BR, 128), lambda s, i: (i, 0))],
        out_specs=pl.BlockSpec((BR, SW), lambda s, i: (s * NRB + i, 0)),
        out_shape=jax.ShapeDtypeStruct((S * R, SW), jnp.float32),
    )(xp, Wc, dinv)


def _tc_mid(agg, b3, W, dinv):
    def body(a_ref, b_ref, w_ref, d_ref, o_ref):
        k = pl.program_id(2)
        dv = d_ref[:, :1]
        hblk = jnp.maximum(dv * a_ref[...] + b_ref[0], 0.0)
        mm = _dot(hblk, w_ref[...])

        @pl.when(k == 0)
        def _():
            o_ref[...] = mm

        @pl.when(k > 0)
        def _():
            o_ref[...] += mm

        @pl.when(k == S - 1)
        def _():
            o_ref[...] *= dv

    return pl.pallas_call(
        body,
        grid=(S, NRB, S),
        in_specs=[pl.BlockSpec((BR, SW), lambda s, i, k: (k * NRB + i, 0)),
                  pl.BlockSpec((1, 1, SW), lambda s, i, k: (k, 0, 0)),
                  pl.BlockSpec((SW, SW), lambda s, i, k: (k, s)),
                  pl.BlockSpec((BR, 128), lambda s, i, k: (i, 0))],
        out_specs=pl.BlockSpec((BR, SW), lambda s, i, k: (s * NRB + i, 0)),
        out_shape=jax.ShapeDtypeStruct((S * R, SW), jnp.float32),
    )(agg, b3, W, dinv)


def _tc_cls(agg, b3, Wout, dinv):
    def body(a_ref, b_ref, w_ref, d_ref, o_ref):
        k = pl.program_id(1)
        dv = d_ref[:, :1]
        hblk = jnp.maximum(dv * a_ref[...] + b_ref[0], 0.0)
        mm = _dot(hblk, w_ref[...])

        @pl.when(k == 0)
        def _():
            o_ref[...] = mm

        @pl.when(k > 0)
        def _():
            o_ref[...] += mm

        @pl.when(k == S - 1)
        def _():
            z = o_ref[...]
            m = jnp.max(z, axis=1, keepdims=True)
            ez = jnp.exp(z - m)
            o_ref[...] = (z - m) - jnp.log(jnp.sum(ez, axis=1, keepdims=True))

    return pl.pallas_call(
        body,
        grid=(NRB, S),
        in_specs=[pl.BlockSpec((BR, SW), lambda i, k: (k * NRB + i, 0)),
                  pl.BlockSpec((1, 1, SW), lambda i, k: (k, 0, 0)),
                  pl.BlockSpec((SW, C), lambda i, k: (k, 0)),
                  pl.BlockSpec((BR, 128), lambda i, k: (i, 0))],
        out_specs=pl.BlockSpec((BR, C), lambda i, k: (i, 0)),
        out_shape=jax.ShapeDtypeStruct((R, C), jnp.float32),
    )(agg, b3, Wout, dinv)


def kernel(x, edge_index, weight_in, weight_out, W0, W1, W2, b0, b1, b2):
    i32 = jnp.int32
    f32 = jnp.float32
    src = edge_index[0]
    dst = edge_index[1]

    pad = jnp.full((EPAD - E,), N, i32)
    srcp = jnp.concatenate([src, pad]).reshape(NT, NCH, ECH)
    dstp = jnp.concatenate([dst, pad]).reshape(NT, NCH, ECH)
    src_slab = srcp[None] + (jnp.arange(S, dtype=i32) * R)[:, None, None, None]
    dst2 = jnp.concatenate(
        [dst, jnp.full((EPAD2 - E,), N, i32)]).reshape(2, NT, NCH2, ECH)
    xp = jnp.pad(x, ((0, R - N), (0, 0)))
    zeros128 = jnp.zeros((R, 128), f32)
    ones128 = jnp.ones((ECH, 128), f32)

    degp = _sc_degree(dst2, zeros128, ones128)
    dinv = _tc_dinv(degp)
    Wc = _tc_wprod(weight_in, W0)
    g = _tc_first(xp, Wc, dinv)
    for (W, b) in ((W1, b0), (W2, b1)):
        agg = _sc_aggregate(g, src_slab, dstp)
        g = _tc_mid(agg, b.reshape(S, 1, SW), W, dinv)
    agg = _sc_aggregate(g, src_slab, dstp)
    out = _tc_cls(agg, b2.reshape(S, 1, SW), weight_out, dinv)
    return out[:N]
